# Initial kernel scaffold; baseline (speedup 1.0000x reference)
import jax
import jax.numpy as jnp
from jax.experimental import pallas as pl


def kernel(x, edge_index, batch, W1, b1, W2, b2, F1, fb1, F2, fb2):
    raise NotImplementedError("write your pallas kernel here")


# speedup vs baseline: 17.7195x; 17.7195x over previous
import functools

import jax
import jax.numpy as jnp
from jax import lax
from jax.experimental import pallas as pl
from jax.experimental.pallas import tpu as pltpu
from jax.experimental.pallas import tpu_sc as plsc

NC = 2
NS = 16
NW = NC * NS
DEG_W = 128
K = 80
ZROWS = 128
SPAN = 5 * ZROWS
STRIDE = 624


def _sc_deg(dst, n, e):
  ew = e // NW
  nch = ew // K
  assert (NS - 1) * STRIDE + SPAN == n
  nzc = SPAN // ZROWS
  mesh = plsc.VectorSubcoreMesh(core_axis_name="c", subcore_axis_name="s")

  @functools.partial(
      pl.kernel,
      out_type=jax.ShapeDtypeStruct((NC * n, DEG_W), jnp.float32),
      mesh=mesh,
      scratch_types=[
          pltpu.VMEM((K, DEG_W), jnp.float32),
          pltpu.VMEM((1, K), jnp.int32),
          pltpu.VMEM((ZROWS, DEG_W), jnp.float32),
          pltpu.VMEM_SHARED((n, DEG_W), jnp.float32),
      ],
  )
  def k(dst_hbm, out_hbm, ones_v, di_v, z_v, acc_sh):
    c = lax.axis_index("c")
    s = lax.axis_index("s")
    wid = c * NS + s
    ebase = wid * ew
    row0 = s * STRIDE

    nl = DEG_W // 16

    def fill_ones(i, _):
      ones_v[i // nl, pl.ds((i % nl) * 16, 16)] = jnp.ones((16,), jnp.float32)
      return 0
    lax.fori_loop(0, K * nl, fill_ones, 0)

    def fill_zero(i, _):
      z_v[i // nl, pl.ds((i % nl) * 16, 16)] = jnp.zeros((16,), jnp.float32)
      return 0
    lax.fori_loop(0, ZROWS * nl, fill_zero, 0)

    for i in range(nzc):
      pltpu.sync_copy(z_v, acc_sh.at[pl.ds(row0 + i * ZROWS, ZROWS)])
    plsc.subcore_barrier()

    def body(j, _):
      pltpu.sync_copy(dst_hbm.at[pl.ds(ebase + j * K, K)], di_v.at[0])
      pltpu.sync_copy(ones_v, acc_sh.at[di_v.at[0]], add=True)
      return 0
    lax.fori_loop(0, nch, body, 0)
    plsc.subcore_barrier()

    for i in range(nzc):
      sl = pl.ds(row0 + i * ZROWS, ZROWS)
      pltpu.sync_copy(acc_sh.at[sl],
                      out_hbm.at[pl.ds(c * n + row0 + i * ZROWS, ZROWS)])

  return k(dst)


def _sc_agg(h, src, dst, n, e, d):
  ew = e // NW
  nch = ew // K
  assert (NS - 1) * STRIDE + SPAN == n
  nzc = SPAN // ZROWS
  mesh = plsc.VectorSubcoreMesh(core_axis_name="c", subcore_axis_name="s")

  @functools.partial(
      pl.kernel,
      out_type=jax.ShapeDtypeStruct((NC * n, d), jnp.float32),
      mesh=mesh,
      scratch_types=[
          pltpu.VMEM((2, K), jnp.int32),
          pltpu.VMEM((2, K), jnp.int32),
          pltpu.VMEM((2, K, d), jnp.float32),
          pltpu.VMEM((ZROWS, d), jnp.float32),
          pltpu.VMEM_SHARED((n, d), jnp.float32),
          pltpu.SemaphoreType.DMA,
          pltpu.SemaphoreType.DMA,
      ],
  )
  def k(h_hbm, src_hbm, dst_hbm, out_hbm,
        si_v, di_v, rows_v, z_v, acc_sh, sem0, sem1):
    c = lax.axis_index("c")
    s = lax.axis_index("s")
    wid = c * NS + s
    ebase = wid * ew
    row0 = s * STRIDE
    nl = d // 16

    def fill_zero(i, _):
      z_v[i // nl, pl.ds((i % nl) * 16, 16)] = jnp.zeros((16,), jnp.float32)
      return 0
    lax.fori_loop(0, ZROWS * nl, fill_zero, 0)

    for i in range(nzc):
      pltpu.sync_copy(z_v, acc_sh.at[pl.ds(row0 + i * ZROWS, ZROWS)])
    plsc.subcore_barrier()

    sems = (sem0, sem1)

    def load_idx(j, b):
      off = ebase + j * K
      pltpu.sync_copy(src_hbm.at[pl.ds(off, K)], si_v.at[b])
      pltpu.sync_copy(dst_hbm.at[pl.ds(off, K)], di_v.at[b])

    def start_gather(b):
      pltpu.async_copy(h_hbm.at[si_v.at[b]], rows_v.at[b], sems[b])

    def wait_gather(b):
      pltpu.make_async_copy(h_hbm.at[si_v.at[b]], rows_v.at[b],
                            sems[b]).wait()

    def scat(b):
      pltpu.sync_copy(rows_v.at[b], acc_sh.at[di_v.at[b]], add=True)

    load_idx(0, 0)
    start_gather(0)

    def pair(g, _):
      load_idx(2 * g + 1, 1)
      start_gather(1)
      wait_gather(0)
      scat(0)
      load_idx(2 * g + 2, 0)
      start_gather(0)
      wait_gather(1)
      scat(1)
      return 0
    lax.fori_loop(0, (nch - 1) // 2, pair, 0)
    wait_gather(0)
    scat(0)
    plsc.subcore_barrier()

    for i in range(nzc):
      sl = pl.ds(row0 + i * ZROWS, ZROWS)
      pltpu.sync_copy(acc_sh.at[sl],
                      out_hbm.at[pl.ds(c * n + row0 + i * ZROWS, ZROWS)])

  return k(h, src, dst)


def _dinv_blk(d0_ref, d1_ref):
  deg = d0_ref[:, 0:1] + d1_ref[:, 0:1] + 1.0
  return lax.rsqrt(deg)


def _tc_pre(x, w1, degp, n, bn):
  ng = n // bn
  nb = ng

  def body(x_ref, w_ref, d0_ref, d1_ref, o_ref):
    dinv = _dinv_blk(d0_ref, d1_ref)
    o_ref[...] = jnp.dot(x_ref[...], w_ref[...],
                         preferred_element_type=jnp.float32,
                         precision=lax.Precision.HIGHEST) * dinv

  return pl.pallas_call(
      body,
      grid=(ng,),
      in_specs=[
          pl.BlockSpec((bn, x.shape[1]), lambda i: (i, 0)),
          pl.BlockSpec(w1.shape, lambda i: (0, 0)),
          pl.BlockSpec((bn, DEG_W), lambda i: (i, 0)),
          pl.BlockSpec((bn, DEG_W), lambda i: (i + nb, 0)),
      ],
      out_specs=pl.BlockSpec((bn, w1.shape[1]), lambda i: (i, 0)),
      out_shape=jax.ShapeDtypeStruct((n, w1.shape[1]), jnp.float32),
  )(x, w1, degp, degp)


def _tc_mid(agg, h1p, degp, w2, b1, n, bn):
  ng = n // bn
  nb = ng
  h = h1p.shape[1]

  def body(a0_ref, a1_ref, hp_ref, d0_ref, d1_ref, w_ref, b_ref, o_ref):
    dinv = _dinv_blk(d0_ref, d1_ref)
    h1 = (a0_ref[...] + a1_ref[...] + hp_ref[...]) * dinv + b_ref[...]
    h1 = jnp.maximum(h1, 0.0)
    o_ref[...] = jnp.dot(h1, w_ref[...],
                         preferred_element_type=jnp.float32,
                         precision=lax.Precision.HIGHEST) * dinv

  return pl.pallas_call(
      body,
      grid=(ng,),
      in_specs=[
          pl.BlockSpec((bn, h), lambda i: (i, 0)),
          pl.BlockSpec((bn, h), lambda i: (i + nb, 0)),
          pl.BlockSpec((bn, h), lambda i: (i, 0)),
          pl.BlockSpec((bn, DEG_W), lambda i: (i, 0)),
          pl.BlockSpec((bn, DEG_W), lambda i: (i + nb, 0)),
          pl.BlockSpec(w2.shape, lambda i: (0, 0)),
          pl.BlockSpec((1, h), lambda i: (0, 0)),
      ],
      out_specs=pl.BlockSpec((bn, h), lambda i: (i, 0)),
      out_shape=jax.ShapeDtypeStruct((n, h), jnp.float32),
  )(agg, agg, h1p, degp, degp, w2, b1.reshape(1, h))


def _tc_post(agg, h2p, degp, b2, batch3, f1, fb1, f2, fb2, n, bn, g):
  ng = n // bn
  nb = ng
  h = h2p.shape[1]
  out = f2.shape[1]

  def body(a0_ref, a1_ref, hp_ref, d0_ref, d1_ref, b_ref, bt_ref,
           f1_ref, fb1_ref, f2_ref, fb2_ref, z_ref, sums, cnt):
    i = pl.program_id(0)
    dinv = _dinv_blk(d0_ref, d1_ref)
    h2 = (a0_ref[...] + a1_ref[...] + hp_ref[...]) * dinv + b_ref[...]
    bt = bt_ref[0]
    gids = lax.broadcasted_iota(jnp.int32, (g, bn), 0)
    oh = jnp.where(bt == gids, 1.0, 0.0).astype(jnp.float32)
    part = jnp.dot(oh, h2, preferred_element_type=jnp.float32,
                   precision=lax.Precision.HIGHEST)
    pcnt = jnp.sum(oh, axis=1, keepdims=True)

    @pl.when(i == 0)
    def _():
      sums[...] = part
      cnt[...] = pcnt

    @pl.when(i > 0)
    def _():
      sums[...] += part
      cnt[...] += pcnt

    @pl.when(i == ng - 1)
    def _():
      gr = jnp.maximum(sums[...] / jnp.maximum(cnt[...], 1.0), 0.0)
      z1 = jnp.dot(gr, f1_ref[...], preferred_element_type=jnp.float32,
                   precision=lax.Precision.HIGHEST) + fb1_ref[...]
      z1 = jnp.maximum(z1, 0.0)
      z_ref[...] = jnp.dot(z1, f2_ref[...],
                           preferred_element_type=jnp.float32,
                           precision=lax.Precision.HIGHEST) + fb2_ref[...]

  return pl.pallas_call(
      body,
      grid=(ng,),
      in_specs=[
          pl.BlockSpec((bn, h), lambda i: (i, 0)),
          pl.BlockSpec((bn, h), lambda i: (i + nb, 0)),
          pl.BlockSpec((bn, h), lambda i: (i, 0)),
          pl.BlockSpec((bn, DEG_W), lambda i: (i, 0)),
          pl.BlockSpec((bn, DEG_W), lambda i: (i + nb, 0)),
          pl.BlockSpec((1, h), lambda i: (0, 0)),
          pl.BlockSpec((1, 1, bn), lambda i: (i, 0, 0)),
          pl.BlockSpec(f1.shape, lambda i: (0, 0)),
          pl.BlockSpec((1, h), lambda i: (0, 0)),
          pl.BlockSpec(f2.shape, lambda i: (0, 0)),
          pl.BlockSpec((1, out), lambda i: (0, 0)),
      ],
      out_specs=pl.BlockSpec((g, out), lambda i: (0, 0)),
      out_shape=jax.ShapeDtypeStruct((g, out), jnp.float32),
      scratch_shapes=[
          pltpu.VMEM((g, h), jnp.float32),
          pltpu.VMEM((g, 1), jnp.float32),
      ],
  )(agg, agg, h2p, degp, degp, b2.reshape(1, h), batch3,
    f1, fb1.reshape(1, h), f2, fb2.reshape(1, out))


def kernel(x, edge_index, batch, W1, b1, W2, b2, F1, fb1, F2, fb2):
  n, d = x.shape
  e = edge_index.shape[1]
  g = 128
  bn = 1000
  src = edge_index[0]
  dst = edge_index[1]
  batch3 = batch.reshape(n // bn, 1, bn)

  degp = _sc_deg(dst, n, e)
  h1p = _tc_pre(x, W1, degp, n, bn)
  a1 = _sc_agg(h1p, src, dst, n, e, d)
  h2p = _tc_mid(a1, h1p, degp, W2, b1, n, bn)
  a2 = _sc_agg(h2p, src, dst, n, e, d)
  z = _tc_post(a2, h2p, degp, b2, batch3, F1, fb1, F2, fb2, n, bn, g)
  return z


# speedup vs baseline: 22.7841x; 1.2858x over previous
import functools

import jax
import jax.numpy as jnp
from jax import lax
from jax.experimental import pallas as pl
from jax.experimental.pallas import tpu as pltpu
from jax.experimental.pallas import tpu_sc as plsc

NC = 2
NS = 16
NW = NC * NS
DEG_W = 128
K = 80
ZROWS = 128
ZINIT = 64
SPAN = 5 * ZROWS
STRIDE = 624


def _sc_deg(dst, n, e):
  ew = e // NW
  nch = ew // K
  assert (NS - 1) * STRIDE + SPAN == n
  nzc = SPAN // ZROWS
  mesh = plsc.VectorSubcoreMesh(core_axis_name="c", subcore_axis_name="s")

  @functools.partial(
      pl.kernel,
      out_type=jax.ShapeDtypeStruct((NC * n, DEG_W), jnp.float32),
      mesh=mesh,
      scratch_types=[
          pltpu.VMEM((K, DEG_W), jnp.float32),
          pltpu.VMEM((2, K), jnp.int32),
          pltpu.VMEM((ZINIT, DEG_W), jnp.float32),
          pltpu.VMEM_SHARED((n, DEG_W), jnp.float32),
          pltpu.SemaphoreType.DMA,
          pltpu.SemaphoreType.DMA,
      ],
  )
  def k(dst_hbm, out_hbm, ones_v, di_v, z_v, acc_sh, sem0, sem1):
    c = lax.axis_index("c")
    s = lax.axis_index("s")
    wid = c * NS + s
    ebase = wid * ew
    row0 = s * STRIDE
    wide = DEG_W // 16

    def fill_ones(i, _):
      ones_v[i // wide, pl.ds((i % wide) * 16, 16)] = jnp.ones(
          (16,), jnp.float32)
      return 0
    lax.fori_loop(0, K * wide, fill_ones, 0)

    def fill_z(i, _):
      z_v[i // wide, pl.ds((i % wide) * 16, 16)] = jnp.zeros(
          (16,), jnp.float32)
      return 0
    lax.fori_loop(0, ZINIT * wide, fill_z, 0)

    for i in range(SPAN // ZINIT):
      pltpu.sync_copy(z_v, acc_sh.at[pl.ds(row0 + i * ZINIT, ZINIT)])
    plsc.subcore_barrier()

    sems = (sem0, sem1)

    def load_idx(j, b):
      pltpu.sync_copy(dst_hbm.at[pl.ds(ebase + j * K, K)], di_v.at[b])

    def start_scat(b):
      pltpu.async_copy(ones_v, acc_sh.at[di_v.at[b]], sems[b], add=True)

    def wait_scat(b):
      pltpu.make_async_copy(ones_v, acc_sh.at[di_v.at[b]], sems[b]).wait()

    load_idx(0, 0)
    start_scat(0)

    def body(g, _):
      load_idx(2 * g + 1, 1)
      start_scat(1)
      wait_scat(0)
      load_idx(2 * g + 2, 0)
      start_scat(0)
      wait_scat(1)
      return 0
    lax.fori_loop(0, (nch - 1) // 2, body, 0)
    wait_scat(0)
    plsc.subcore_barrier()

    for i in range(nzc):
      base = row0 + i * ZROWS
      pltpu.sync_copy(acc_sh.at[pl.ds(base, ZROWS)],
                      out_hbm.at[pl.ds(c * n + base, ZROWS)])

  return k(dst)


NB = 3


def _sc_agg(h, src, dst, n, e, d):
  ew = e // NW
  nch = ew // K
  assert (NS - 1) * STRIDE + SPAN == n
  nzc = SPAN // ZROWS
  mesh = plsc.VectorSubcoreMesh(core_axis_name="c", subcore_axis_name="s")

  @functools.partial(
      pl.kernel,
      out_type=jax.ShapeDtypeStruct((NC * n, d), jnp.float32),
      mesh=mesh,
      scratch_types=[
          pltpu.VMEM((NB, K), jnp.int32),
          pltpu.VMEM((NB, K), jnp.int32),
          pltpu.VMEM((NB, K, d), jnp.float32),
          pltpu.VMEM((ZINIT, d), jnp.float32),
          pltpu.VMEM_SHARED((n, d), jnp.float32),
          [pltpu.SemaphoreType.DMA] * NB,
          [pltpu.SemaphoreType.DMA] * NB,
      ],
  )
  def k(h_hbm, src_hbm, dst_hbm, out_hbm,
        si_v, di_v, rows_v, z_v, acc_sh, gsems, ssems):
    c = lax.axis_index("c")
    s = lax.axis_index("s")
    wid = c * NS + s
    ebase = wid * ew
    row0 = s * STRIDE
    nl = d // 16

    def fill_zero(i, _):
      z_v[i // nl, pl.ds((i % nl) * 16, 16)] = jnp.zeros((16,), jnp.float32)
      return 0
    lax.fori_loop(0, ZINIT * nl, fill_zero, 0)

    for i in range(SPAN // ZINIT):
      pltpu.sync_copy(z_v, acc_sh.at[pl.ds(row0 + i * ZINIT, ZINIT)])
    plsc.subcore_barrier()

    def load_idx(j, b):
      off = ebase + j * K
      pltpu.sync_copy(src_hbm.at[pl.ds(off, K)], si_v.at[b])
      pltpu.sync_copy(dst_hbm.at[pl.ds(off, K)], di_v.at[b])

    def start_gather(b):
      pltpu.async_copy(h_hbm.at[si_v.at[b]], rows_v.at[b], gsems[b])

    def wait_gather(b):
      pltpu.make_async_copy(h_hbm.at[si_v.at[b]], rows_v.at[b],
                            gsems[b]).wait()

    def start_scat(b):
      pltpu.async_copy(rows_v.at[b], acc_sh.at[di_v.at[b]], ssems[b],
                       add=True)

    def wait_scat(b):
      pltpu.make_async_copy(rows_v.at[b], acc_sh.at[di_v.at[b]],
                            ssems[b]).wait()

    load_idx(0, 0)
    start_gather(0)
    load_idx(1, 1)
    start_gather(1)

    def group(g, _):
      for i in range(NB):
        j = NB * g + i
        b = i
        bprev = (i - 1) % NB
        bnext2 = (i + 2) % NB
        wait_gather(b)
        start_scat(b)

        @pl.when(j >= 1)
        def _():
          wait_scat(bprev)

        @pl.when(j + 2 < nch)
        def _():
          load_idx(jnp.minimum(j + 2, nch - 1), bnext2)
          start_gather(bnext2)
      return 0
    lax.fori_loop(0, nch // NB, group, 0)

    for j in range(NB * (nch // NB), nch):
      b = j % NB
      bprev = (b - 1) % NB
      wait_gather(b)
      start_scat(b)
      wait_scat(bprev)
    wait_scat((nch - 1) % NB)
    plsc.subcore_barrier()

    for i in range(nzc):
      sl = pl.ds(row0 + i * ZROWS, ZROWS)
      pltpu.sync_copy(acc_sh.at[sl],
                      out_hbm.at[pl.ds(c * n + row0 + i * ZROWS, ZROWS)])

  return k(h, src, dst)


def _dinv_blk(d0_ref, d1_ref):
  deg = d0_ref[:, 0:1] + d1_ref[:, 0:1] + 1.0
  return lax.rsqrt(deg)


def _tc_pre(x, w1, degp, n, bn):
  ng = n // bn
  nb = ng

  def body(x_ref, w_ref, d0_ref, d1_ref, o_ref):
    dinv = _dinv_blk(d0_ref, d1_ref)
    o_ref[...] = jnp.dot(x_ref[...], w_ref[...],
                         preferred_element_type=jnp.float32,
                         precision=lax.Precision.HIGHEST) * dinv

  return pl.pallas_call(
      body,
      grid=(ng,),
      in_specs=[
          pl.BlockSpec((bn, x.shape[1]), lambda i: (i, 0)),
          pl.BlockSpec(w1.shape, lambda i: (0, 0)),
          pl.BlockSpec((bn, DEG_W), lambda i: (i, 0)),
          pl.BlockSpec((bn, DEG_W), lambda i: (i + nb, 0)),
      ],
      out_specs=pl.BlockSpec((bn, w1.shape[1]), lambda i: (i, 0)),
      out_shape=jax.ShapeDtypeStruct((n, w1.shape[1]), jnp.float32),
  )(x, w1, degp, degp)


def _tc_mid(agg, h1p, degp, w2, b1, n, bn):
  ng = n // bn
  nb = ng
  h = h1p.shape[1]

  def body(a0_ref, a1_ref, hp_ref, d0_ref, d1_ref, w_ref, b_ref, o_ref):
    dinv = _dinv_blk(d0_ref, d1_ref)
    h1 = (a0_ref[...] + a1_ref[...] + hp_ref[...]) * dinv + b_ref[...]
    h1 = jnp.maximum(h1, 0.0)
    o_ref[...] = jnp.dot(h1, w_ref[...],
                         preferred_element_type=jnp.float32,
                         precision=lax.Precision.HIGHEST) * dinv

  return pl.pallas_call(
      body,
      grid=(ng,),
      in_specs=[
          pl.BlockSpec((bn, h), lambda i: (i, 0)),
          pl.BlockSpec((bn, h), lambda i: (i + nb, 0)),
          pl.BlockSpec((bn, h), lambda i: (i, 0)),
          pl.BlockSpec((bn, DEG_W), lambda i: (i, 0)),
          pl.BlockSpec((bn, DEG_W), lambda i: (i + nb, 0)),
          pl.BlockSpec(w2.shape, lambda i: (0, 0)),
          pl.BlockSpec((1, h), lambda i: (0, 0)),
      ],
      out_specs=pl.BlockSpec((bn, h), lambda i: (i, 0)),
      out_shape=jax.ShapeDtypeStruct((n, h), jnp.float32),
  )(agg, agg, h1p, degp, degp, w2, b1.reshape(1, h))


def _tc_post(agg, h2p, degp, b2, batch3, f1, fb1, f2, fb2, n, bn, g):
  ng = n // bn
  nb = ng
  h = h2p.shape[1]
  out = f2.shape[1]

  def body(a0_ref, a1_ref, hp_ref, d0_ref, d1_ref, b_ref, bt_ref,
           f1_ref, fb1_ref, f2_ref, fb2_ref, z_ref, sums, cnt):
    i = pl.program_id(0)
    dinv = _dinv_blk(d0_ref, d1_ref)
    h2 = (a0_ref[...] + a1_ref[...] + hp_ref[...]) * dinv + b_ref[...]
    bt = bt_ref[0]
    gids = lax.broadcasted_iota(jnp.int32, (g, bn), 0)
    oh = jnp.where(bt == gids, 1.0, 0.0).astype(jnp.float32)
    part = jnp.dot(oh, h2, preferred_element_type=jnp.float32,
                   precision=lax.Precision.HIGHEST)
    pcnt = jnp.sum(oh, axis=1, keepdims=True)

    @pl.when(i == 0)
    def _():
      sums[...] = part
      cnt[...] = pcnt

    @pl.when(i > 0)
    def _():
      sums[...] += part
      cnt[...] += pcnt

    @pl.when(i == ng - 1)
    def _():
      gr = jnp.maximum(sums[...] / jnp.maximum(cnt[...], 1.0), 0.0)
      z1 = jnp.dot(gr, f1_ref[...], preferred_element_type=jnp.float32,
                   precision=lax.Precision.HIGHEST) + fb1_ref[...]
      z1 = jnp.maximum(z1, 0.0)
      z_ref[...] = jnp.dot(z1, f2_ref[...],
                           preferred_element_type=jnp.float32,
                           precision=lax.Precision.HIGHEST) + fb2_ref[...]

  return pl.pallas_call(
      body,
      grid=(ng,),
      in_specs=[
          pl.BlockSpec((bn, h), lambda i: (i, 0)),
          pl.BlockSpec((bn, h), lambda i: (i + nb, 0)),
          pl.BlockSpec((bn, h), lambda i: (i, 0)),
          pl.BlockSpec((bn, DEG_W), lambda i: (i, 0)),
          pl.BlockSpec((bn, DEG_W), lambda i: (i + nb, 0)),
          pl.BlockSpec((1, h), lambda i: (0, 0)),
          pl.BlockSpec((1, 1, bn), lambda i: (i, 0, 0)),
          pl.BlockSpec(f1.shape, lambda i: (0, 0)),
          pl.BlockSpec((1, h), lambda i: (0, 0)),
          pl.BlockSpec(f2.shape, lambda i: (0, 0)),
          pl.BlockSpec((1, out), lambda i: (0, 0)),
      ],
      out_specs=pl.BlockSpec((g, out), lambda i: (0, 0)),
      out_shape=jax.ShapeDtypeStruct((g, out), jnp.float32),
      scratch_shapes=[
          pltpu.VMEM((g, h), jnp.float32),
          pltpu.VMEM((g, 1), jnp.float32),
      ],
  )(agg, agg, h2p, degp, degp, b2.reshape(1, h), batch3,
    f1, fb1.reshape(1, h), f2, fb2.reshape(1, out))


def kernel(x, edge_index, batch, W1, b1, W2, b2, F1, fb1, F2, fb2):
  n, d = x.shape
  e = edge_index.shape[1]
  g = 128
  bn = 1000
  src = edge_index[0]
  dst = edge_index[1]
  batch3 = batch.reshape(n // bn, 1, bn)

  degp = _sc_deg(dst, n, e)
  h1p = _tc_pre(x, W1, degp, n, bn)
  a1 = _sc_agg(h1p, src, dst, n, e, d)
  h2p = _tc_mid(a1, h1p, degp, W2, b1, n, bn)
  a2 = _sc_agg(h2p, src, dst, n, e, d)
  z = _tc_post(a2, h2p, degp, b2, batch3, F1, fb1, F2, fb2, n, bn, g)
  return z


# speedup vs baseline: 29.0594x; 1.2754x over previous
import functools

import jax
import jax.numpy as jnp
from jax import lax
from jax.experimental import pallas as pl
from jax.experimental.pallas import tpu as pltpu
from jax.experimental.pallas import tpu_sc as plsc

NC = 2
NS = 16
NW = NC * NS
DEG_W = 128
K = 80
ZROWS = 128
ZINIT = 64
SPAN = 5 * ZROWS
STRIDE = 624


def _sc_deg(dst, n, e):
  ew = e // NW
  nch = ew // K
  assert (NS - 1) * STRIDE + SPAN == n
  nzc = SPAN // ZROWS
  mesh = plsc.VectorSubcoreMesh(core_axis_name="c", subcore_axis_name="s")

  @functools.partial(
      pl.kernel,
      out_type=jax.ShapeDtypeStruct((NC * n, DEG_W), jnp.float32),
      mesh=mesh,
      scratch_types=[
          pltpu.VMEM((K, DEG_W), jnp.float32),
          pltpu.VMEM((2, K), jnp.int32),
          pltpu.VMEM((ZINIT, DEG_W), jnp.float32),
          pltpu.VMEM_SHARED((n, DEG_W), jnp.float32),
          pltpu.SemaphoreType.DMA,
          pltpu.SemaphoreType.DMA,
      ],
  )
  def k(dst_hbm, out_hbm, ones_v, di_v, z_v, acc_sh, sem0, sem1):
    c = lax.axis_index("c")
    s = lax.axis_index("s")
    wid = c * NS + s
    ebase = wid * ew
    row0 = s * STRIDE
    wide = DEG_W // 16

    def fill_ones(i, _):
      ones_v[i // wide, pl.ds((i % wide) * 16, 16)] = jnp.ones(
          (16,), jnp.float32)
      return 0
    lax.fori_loop(0, K * wide, fill_ones, 0)

    def fill_z(i, _):
      z_v[i // wide, pl.ds((i % wide) * 16, 16)] = jnp.zeros(
          (16,), jnp.float32)
      return 0
    lax.fori_loop(0, ZINIT * wide, fill_z, 0)

    for i in range(SPAN // ZINIT):
      pltpu.sync_copy(z_v, acc_sh.at[pl.ds(row0 + i * ZINIT, ZINIT)])
    plsc.subcore_barrier()

    sems = (sem0, sem1)

    def load_idx(j, b):
      pltpu.sync_copy(dst_hbm.at[pl.ds(ebase + j * K, K)], di_v.at[b])

    def start_scat(b):
      pltpu.async_copy(ones_v, acc_sh.at[di_v.at[b]], sems[b], add=True)

    def wait_scat(b):
      pltpu.make_async_copy(ones_v, acc_sh.at[di_v.at[b]], sems[b]).wait()

    load_idx(0, 0)
    start_scat(0)

    def body(g, _):
      load_idx(2 * g + 1, 1)
      start_scat(1)
      wait_scat(0)
      load_idx(2 * g + 2, 0)
      start_scat(0)
      wait_scat(1)
      return 0
    lax.fori_loop(0, (nch - 1) // 2, body, 0)
    wait_scat(0)
    plsc.subcore_barrier()

    for i in range(nzc):
      base = row0 + i * ZROWS
      pltpu.sync_copy(acc_sh.at[pl.ds(base, ZROWS)],
                      out_hbm.at[pl.ds(c * n + base, ZROWS)])

  return k(dst)


NB = 3
NI = 6


def _sc_agg(h, src, dst, n, e, d):
  ew = e // NW
  nch = ew // K
  assert (NS - 1) * STRIDE + SPAN == n
  nzc = SPAN // ZROWS
  mesh = plsc.VectorSubcoreMesh(core_axis_name="c", subcore_axis_name="s")

  @functools.partial(
      pl.kernel,
      out_type=jax.ShapeDtypeStruct((NC * n, d), jnp.float32),
      mesh=mesh,
      scratch_types=[
          pltpu.VMEM((NI, K), jnp.int32),
          pltpu.VMEM((NI, K), jnp.int32),
          pltpu.VMEM((NB, K, d), jnp.float32),
          pltpu.VMEM((ZINIT, d), jnp.float32),
          pltpu.VMEM_SHARED((n, d), jnp.float32),
          [pltpu.SemaphoreType.DMA] * NI,
          [pltpu.SemaphoreType.DMA] * NB,
          [pltpu.SemaphoreType.DMA] * NB,
      ],
  )
  def k(h_hbm, src_hbm, dst_hbm, out_hbm,
        si_v, di_v, rows_v, z_v, acc_sh, isems, gsems, ssems):
    c = lax.axis_index("c")
    s = lax.axis_index("s")
    wid = c * NS + s
    ebase = wid * ew
    row0 = s * STRIDE
    nl = d // 16

    def fill_zero(i, _):
      z_v[i // nl, pl.ds((i % nl) * 16, 16)] = jnp.zeros((16,), jnp.float32)
      return 0
    lax.fori_loop(0, ZINIT * nl, fill_zero, 0)

    for i in range(SPAN // ZINIT):
      pltpu.sync_copy(z_v, acc_sh.at[pl.ds(row0 + i * ZINIT, ZINIT)])
    plsc.subcore_barrier()

    def start_idx(j, q):
      off = ebase + j * K
      pltpu.async_copy(src_hbm.at[pl.ds(off, K)], si_v.at[q], isems[q])
      pltpu.async_copy(dst_hbm.at[pl.ds(off, K)], di_v.at[q], isems[q])

    def wait_idx(q):
      pltpu.make_async_copy(src_hbm.at[pl.ds(ebase, K)], si_v.at[q],
                            isems[q]).wait()
      pltpu.make_async_copy(dst_hbm.at[pl.ds(ebase, K)], di_v.at[q],
                            isems[q]).wait()

    def start_gather(b, q):
      pltpu.async_copy(h_hbm.at[si_v.at[q]], rows_v.at[b], gsems[b])

    def wait_gather(b, q):
      pltpu.make_async_copy(h_hbm.at[si_v.at[q]], rows_v.at[b],
                            gsems[b]).wait()

    def start_scat(b, q):
      pltpu.async_copy(rows_v.at[b], acc_sh.at[di_v.at[q]], ssems[b],
                       add=True)

    def wait_scat(b, q):
      pltpu.make_async_copy(rows_v.at[b], acc_sh.at[di_v.at[q]],
                            ssems[b]).wait()

    for j in range(NB + 1):
      start_idx(j, j)
    for j in range(2):
      wait_idx(j)
      start_gather(j, j)

    def chunk_ops(j, i, dyn):
      b = i % NB
      q = i % NI
      bprev = (i - 1) % NB
      qprev = (i - 1) % NI
      b2 = (i + 2) % NB
      q2 = (i + 2) % NI
      q4 = (i + 4) % NI
      wait_gather(b, q)
      start_scat(b, q)

      def retire_prev():
        wait_scat(bprev, qprev)

      def fetch_idx():
        start_idx(jnp.minimum(j + 4, nch - 1) if dyn else j + 4, q4)

      def next_gather():
        wait_idx(q2)
        start_gather(b2, q2)

      if dyn:
        pl.when(j >= 1)(retire_prev)
        pl.when(j + 4 < nch)(fetch_idx)
        pl.when(j + 2 < nch)(next_gather)
      else:
        if j >= 1:
          retire_prev()
        if j + 4 < nch:
          fetch_idx()
        if j + 2 < nch:
          next_gather()

    nun = NI * (nch // NI)

    def group(g, _):
      for i in range(NI):
        chunk_ops(NI * g + i, i, True)
      return 0
    lax.fori_loop(0, nch // NI, group, 0)
    for j in range(nun, nch):
      chunk_ops(j, j, False)
    wait_scat((nch - 1) % NB, (nch - 1) % NI)
    plsc.subcore_barrier()

    for i in range(nzc):
      sl = pl.ds(row0 + i * ZROWS, ZROWS)
      pltpu.sync_copy(acc_sh.at[sl],
                      out_hbm.at[pl.ds(c * n + row0 + i * ZROWS, ZROWS)])

  return k(h, src, dst)


def _dinv_blk(d0_ref, d1_ref):
  deg = d0_ref[:, 0:1] + d1_ref[:, 0:1] + 1.0
  return lax.rsqrt(deg)


def _tc_pre(x, w1, degp, n, bn):
  ng = n // bn
  nb = ng

  def body(x_ref, w_ref, d0_ref, d1_ref, o_ref):
    dinv = _dinv_blk(d0_ref, d1_ref)
    o_ref[...] = jnp.dot(x_ref[...], w_ref[...],
                         preferred_element_type=jnp.float32,
                         precision=lax.Precision.HIGHEST) * dinv

  return pl.pallas_call(
      body,
      grid=(ng,),
      in_specs=[
          pl.BlockSpec((bn, x.shape[1]), lambda i: (i, 0)),
          pl.BlockSpec(w1.shape, lambda i: (0, 0)),
          pl.BlockSpec((bn, DEG_W), lambda i: (i, 0)),
          pl.BlockSpec((bn, DEG_W), lambda i: (i + nb, 0)),
      ],
      out_specs=pl.BlockSpec((bn, w1.shape[1]), lambda i: (i, 0)),
      out_shape=jax.ShapeDtypeStruct((n, w1.shape[1]), jnp.float32),
  )(x, w1, degp, degp)


def _tc_mid(agg, h1p, degp, w2, b1, n, bn):
  ng = n // bn
  nb = ng
  h = h1p.shape[1]

  def body(a0_ref, a1_ref, hp_ref, d0_ref, d1_ref, w_ref, b_ref, o_ref):
    dinv = _dinv_blk(d0_ref, d1_ref)
    h1 = (a0_ref[...] + a1_ref[...] + hp_ref[...]) * dinv + b_ref[...]
    h1 = jnp.maximum(h1, 0.0)
    o_ref[...] = jnp.dot(h1, w_ref[...],
                         preferred_element_type=jnp.float32,
                         precision=lax.Precision.HIGHEST) * dinv

  return pl.pallas_call(
      body,
      grid=(ng,),
      in_specs=[
          pl.BlockSpec((bn, h), lambda i: (i, 0)),
          pl.BlockSpec((bn, h), lambda i: (i + nb, 0)),
          pl.BlockSpec((bn, h), lambda i: (i, 0)),
          pl.BlockSpec((bn, DEG_W), lambda i: (i, 0)),
          pl.BlockSpec((bn, DEG_W), lambda i: (i + nb, 0)),
          pl.BlockSpec(w2.shape, lambda i: (0, 0)),
          pl.BlockSpec((1, h), lambda i: (0, 0)),
      ],
      out_specs=pl.BlockSpec((bn, h), lambda i: (i, 0)),
      out_shape=jax.ShapeDtypeStruct((n, h), jnp.float32),
  )(agg, agg, h1p, degp, degp, w2, b1.reshape(1, h))


def _tc_post(agg, h2p, degp, b2, batch3, f1, fb1, f2, fb2, n, bn, g):
  ng = n // bn
  nb = ng
  h = h2p.shape[1]
  out = f2.shape[1]

  def body(a0_ref, a1_ref, hp_ref, d0_ref, d1_ref, b_ref, bt_ref,
           f1_ref, fb1_ref, f2_ref, fb2_ref, z_ref, sums, cnt):
    i = pl.program_id(0)
    dinv = _dinv_blk(d0_ref, d1_ref)
    h2 = (a0_ref[...] + a1_ref[...] + hp_ref[...]) * dinv + b_ref[...]
    bt = bt_ref[0]
    gids = lax.broadcasted_iota(jnp.int32, (g, bn), 0)
    oh = jnp.where(bt == gids, 1.0, 0.0).astype(jnp.float32)
    part = jnp.dot(oh, h2, preferred_element_type=jnp.float32,
                   precision=lax.Precision.HIGHEST)
    pcnt = jnp.sum(oh, axis=1, keepdims=True)

    @pl.when(i == 0)
    def _():
      sums[...] = part
      cnt[...] = pcnt

    @pl.when(i > 0)
    def _():
      sums[...] += part
      cnt[...] += pcnt

    @pl.when(i == ng - 1)
    def _():
      gr = jnp.maximum(sums[...] / jnp.maximum(cnt[...], 1.0), 0.0)
      z1 = jnp.dot(gr, f1_ref[...], preferred_element_type=jnp.float32,
                   precision=lax.Precision.HIGHEST) + fb1_ref[...]
      z1 = jnp.maximum(z1, 0.0)
      z_ref[...] = jnp.dot(z1, f2_ref[...],
                           preferred_element_type=jnp.float32,
                           precision=lax.Precision.HIGHEST) + fb2_ref[...]

  return pl.pallas_call(
      body,
      grid=(ng,),
      in_specs=[
          pl.BlockSpec((bn, h), lambda i: (i, 0)),
          pl.BlockSpec((bn, h), lambda i: (i + nb, 0)),
          pl.BlockSpec((bn, h), lambda i: (i, 0)),
          pl.BlockSpec((bn, DEG_W), lambda i: (i, 0)),
          pl.BlockSpec((bn, DEG_W), lambda i: (i + nb, 0)),
          pl.BlockSpec((1, h), lambda i: (0, 0)),
          pl.BlockSpec((1, 1, bn), lambda i: (i, 0, 0)),
          pl.BlockSpec(f1.shape, lambda i: (0, 0)),
          pl.BlockSpec((1, h), lambda i: (0, 0)),
          pl.BlockSpec(f2.shape, lambda i: (0, 0)),
          pl.BlockSpec((1, out), lambda i: (0, 0)),
      ],
      out_specs=pl.BlockSpec((g, out), lambda i: (0, 0)),
      out_shape=jax.ShapeDtypeStruct((g, out), jnp.float32),
      scratch_shapes=[
          pltpu.VMEM((g, h), jnp.float32),
          pltpu.VMEM((g, 1), jnp.float32),
      ],
  )(agg, agg, h2p, degp, degp, b2.reshape(1, h), batch3,
    f1, fb1.reshape(1, h), f2, fb2.reshape(1, out))


def kernel(x, edge_index, batch, W1, b1, W2, b2, F1, fb1, F2, fb2):
  n, d = x.shape
  e = edge_index.shape[1]
  g = 128
  bn = 1000
  src = edge_index[0]
  dst = edge_index[1]
  batch3 = batch.reshape(n // bn, 1, bn)

  degp = _sc_deg(dst, n, e)
  h1p = _tc_pre(x, W1, degp, n, bn)
  a1 = _sc_agg(h1p, src, dst, n, e, d)
  h2p = _tc_mid(a1, h1p, degp, W2, b1, n, bn)
  a2 = _sc_agg(h2p, src, dst, n, e, d)
  z = _tc_post(a2, h2p, degp, b2, batch3, F1, fb1, F2, fb2, n, bn, g)
  return z


# speedup vs baseline: 30.0265x; 1.0333x over previous
import functools

import jax
import jax.numpy as jnp
from jax import lax
from jax.experimental import pallas as pl
from jax.experimental.pallas import tpu as pltpu
from jax.experimental.pallas import tpu_sc as plsc

NC = 2
NS = 16
NW = NC * NS
DEG_W = 128
K = 80
ZROWS = 128
ZINIT = 64
SPAN = 5 * ZROWS
STRIDE = 624


def _sc_deg(dst, n, e):
  ew = e // NW
  nch = ew // K
  assert (NS - 1) * STRIDE + SPAN == n
  nzc = SPAN // ZROWS
  mesh = plsc.VectorSubcoreMesh(core_axis_name="c", subcore_axis_name="s")

  @functools.partial(
      pl.kernel,
      out_type=jax.ShapeDtypeStruct((NC * n, DEG_W), jnp.float32),
      mesh=mesh,
      scratch_types=[
          pltpu.VMEM((K, DEG_W), jnp.float32),
          pltpu.VMEM((4, K), jnp.int32),
          pltpu.VMEM((ZINIT, DEG_W), jnp.float32),
          pltpu.VMEM_SHARED((n, DEG_W), jnp.float32),
          [pltpu.SemaphoreType.DMA] * 4,
          pltpu.SemaphoreType.DMA,
          pltpu.SemaphoreType.DMA,
      ],
  )
  def k(dst_hbm, out_hbm, ones_v, di_v, z_v, acc_sh, isems, sem0, sem1):
    c = lax.axis_index("c")
    s = lax.axis_index("s")
    wid = c * NS + s
    ebase = wid * ew
    row0 = s * STRIDE
    wide = DEG_W // 16

    def fill_ones(i, _):
      ones_v[i // wide, pl.ds((i % wide) * 16, 16)] = jnp.ones(
          (16,), jnp.float32)
      return 0
    lax.fori_loop(0, K * wide, fill_ones, 0)

    def fill_z(i, _):
      z_v[i // wide, pl.ds((i % wide) * 16, 16)] = jnp.zeros(
          (16,), jnp.float32)
      return 0
    lax.fori_loop(0, ZINIT * wide, fill_z, 0)

    for i in range(SPAN // ZINIT):
      pltpu.sync_copy(z_v, acc_sh.at[pl.ds(row0 + i * ZINIT, ZINIT)])
    plsc.subcore_barrier()

    sems = (sem0, sem1)

    def start_idx(j, q):
      pltpu.async_copy(dst_hbm.at[pl.ds(ebase + j * K, K)], di_v.at[q],
                       isems[q])

    def wait_idx(q):
      pltpu.make_async_copy(dst_hbm.at[pl.ds(ebase, K)], di_v.at[q],
                            isems[q]).wait()

    def start_scat(b, q):
      pltpu.async_copy(ones_v, acc_sh.at[di_v.at[q]], sems[b], add=True)

    def wait_scat(b, q):
      pltpu.make_async_copy(ones_v, acc_sh.at[di_v.at[q]], sems[b]).wait()

    def chunk_ops(j, i, dyn):
      b = i % 2
      q = i % 4
      bprev = (i - 1) % 2
      qprev = (i - 1) % 4
      wait_idx(q)
      start_scat(b, q)

      def retire_prev():
        wait_scat(bprev, qprev)

      def fetch_idx():
        start_idx(jnp.minimum(j + 3, nch - 1) if dyn else j + 3, qprev)

      if dyn:
        pl.when(j >= 1)(retire_prev)
        pl.when(j + 3 < nch)(fetch_idx)
      else:
        if j >= 1:
          retire_prev()
        if j + 3 < nch:
          fetch_idx()

    for j in range(3):
      start_idx(j, j)

    def body(g, _):
      for i in range(4):
        chunk_ops(4 * g + i, i, True)
      return 0
    lax.fori_loop(0, nch // 4, body, 0)
    for j in range(4 * (nch // 4), nch):
      chunk_ops(j, j, False)
    wait_scat((nch - 1) % 2, (nch - 1) % 4)
    plsc.subcore_barrier()

    for i in range(nzc):
      base = row0 + i * ZROWS
      pltpu.sync_copy(acc_sh.at[pl.ds(base, ZROWS)],
                      out_hbm.at[pl.ds(c * n + base, ZROWS)])

  return k(dst)


NB = 3
NI = 6


def _sc_agg(h, src, dst, n, e, d):
  ew = e // NW
  nch = ew // K
  assert (NS - 1) * STRIDE + SPAN == n
  nzc = SPAN // ZROWS
  mesh = plsc.VectorSubcoreMesh(core_axis_name="c", subcore_axis_name="s")

  @functools.partial(
      pl.kernel,
      out_type=jax.ShapeDtypeStruct((NC * n, d), jnp.float32),
      mesh=mesh,
      scratch_types=[
          pltpu.VMEM((NI, K), jnp.int32),
          pltpu.VMEM((NI, K), jnp.int32),
          pltpu.VMEM((NB, K, d), jnp.float32),
          pltpu.VMEM((ZINIT, d), jnp.float32),
          pltpu.VMEM_SHARED((n, d), jnp.float32),
          [pltpu.SemaphoreType.DMA] * NI,
          [pltpu.SemaphoreType.DMA] * NB,
          [pltpu.SemaphoreType.DMA] * NB,
      ],
  )
  def k(h_hbm, src_hbm, dst_hbm, out_hbm,
        si_v, di_v, rows_v, z_v, acc_sh, isems, gsems, ssems):
    c = lax.axis_index("c")
    s = lax.axis_index("s")
    wid = c * NS + s
    ebase = wid * ew
    row0 = s * STRIDE
    nl = d // 16

    def fill_zero(i, _):
      z_v[i // nl, pl.ds((i % nl) * 16, 16)] = jnp.zeros((16,), jnp.float32)
      return 0
    lax.fori_loop(0, ZINIT * nl, fill_zero, 0)

    for i in range(SPAN // ZINIT):
      pltpu.sync_copy(z_v, acc_sh.at[pl.ds(row0 + i * ZINIT, ZINIT)])
    plsc.subcore_barrier()

    def start_idx(j, q):
      off = ebase + j * K
      pltpu.async_copy(src_hbm.at[pl.ds(off, K)], si_v.at[q], isems[q])
      pltpu.async_copy(dst_hbm.at[pl.ds(off, K)], di_v.at[q], isems[q])

    def wait_idx(q):
      pltpu.make_async_copy(src_hbm.at[pl.ds(ebase, K)], si_v.at[q],
                            isems[q]).wait()
      pltpu.make_async_copy(dst_hbm.at[pl.ds(ebase, K)], di_v.at[q],
                            isems[q]).wait()

    def start_gather(b, q):
      pltpu.async_copy(h_hbm.at[si_v.at[q]], rows_v.at[b], gsems[b])

    def wait_gather(b, q):
      pltpu.make_async_copy(h_hbm.at[si_v.at[q]], rows_v.at[b],
                            gsems[b]).wait()

    def start_scat(b, q):
      pltpu.async_copy(rows_v.at[b], acc_sh.at[di_v.at[q]], ssems[b],
                       add=True)

    def wait_scat(b, q):
      pltpu.make_async_copy(rows_v.at[b], acc_sh.at[di_v.at[q]],
                            ssems[b]).wait()

    for j in range(NB + 1):
      start_idx(j, j)
    for j in range(2):
      wait_idx(j)
      start_gather(j, j)

    def chunk_ops(j, i, dyn):
      b = i % NB
      q = i % NI
      bprev = (i - 1) % NB
      qprev = (i - 1) % NI
      b2 = (i + 2) % NB
      q2 = (i + 2) % NI
      q4 = (i + 4) % NI
      wait_gather(b, q)
      start_scat(b, q)

      def retire_prev():
        wait_scat(bprev, qprev)

      def fetch_idx():
        start_idx(jnp.minimum(j + 4, nch - 1) if dyn else j + 4, q4)

      def next_gather():
        wait_idx(q2)
        start_gather(b2, q2)

      if dyn:
        pl.when(j >= 1)(retire_prev)
        pl.when(j + 4 < nch)(fetch_idx)
        pl.when(j + 2 < nch)(next_gather)
      else:
        if j >= 1:
          retire_prev()
        if j + 4 < nch:
          fetch_idx()
        if j + 2 < nch:
          next_gather()

    nun = NI * (nch // NI)

    def group(g, _):
      for i in range(NI):
        chunk_ops(NI * g + i, i, True)
      return 0
    lax.fori_loop(0, nch // NI, group, 0)
    for j in range(nun, nch):
      chunk_ops(j, j, False)
    wait_scat((nch - 1) % NB, (nch - 1) % NI)
    plsc.subcore_barrier()

    for i in range(nzc):
      sl = pl.ds(row0 + i * ZROWS, ZROWS)
      pltpu.sync_copy(acc_sh.at[sl],
                      out_hbm.at[pl.ds(c * n + row0 + i * ZROWS, ZROWS)])

  return k(h, src, dst)


def _dinv_blk(d0_ref, d1_ref):
  deg = d0_ref[:, 0:1] + d1_ref[:, 0:1] + 1.0
  return lax.rsqrt(deg)


def _tc_pre(x, w1, degp, n, bn):
  ng = n // bn
  nb = ng

  def body(x_ref, w_ref, d0_ref, d1_ref, o_ref):
    dinv = _dinv_blk(d0_ref, d1_ref)
    o_ref[...] = jnp.dot(x_ref[...], w_ref[...],
                         preferred_element_type=jnp.float32,
                         precision=lax.Precision.HIGHEST) * dinv

  return pl.pallas_call(
      body,
      grid=(ng,),
      in_specs=[
          pl.BlockSpec((bn, x.shape[1]), lambda i: (i, 0)),
          pl.BlockSpec(w1.shape, lambda i: (0, 0)),
          pl.BlockSpec((bn, DEG_W), lambda i: (i, 0)),
          pl.BlockSpec((bn, DEG_W), lambda i: (i + nb, 0)),
      ],
      out_specs=pl.BlockSpec((bn, w1.shape[1]), lambda i: (i, 0)),
      out_shape=jax.ShapeDtypeStruct((n, w1.shape[1]), jnp.float32),
  )(x, w1, degp, degp)


def _tc_mid(agg, h1p, degp, w2, b1, n, bn):
  ng = n // bn
  nb = ng
  h = h1p.shape[1]

  def body(a0_ref, a1_ref, hp_ref, d0_ref, d1_ref, w_ref, b_ref, o_ref):
    dinv = _dinv_blk(d0_ref, d1_ref)
    h1 = (a0_ref[...] + a1_ref[...] + hp_ref[...]) * dinv + b_ref[...]
    h1 = jnp.maximum(h1, 0.0)
    o_ref[...] = jnp.dot(h1, w_ref[...],
                         preferred_element_type=jnp.float32,
                         precision=lax.Precision.HIGHEST) * dinv

  return pl.pallas_call(
      body,
      grid=(ng,),
      in_specs=[
          pl.BlockSpec((bn, h), lambda i: (i, 0)),
          pl.BlockSpec((bn, h), lambda i: (i + nb, 0)),
          pl.BlockSpec((bn, h), lambda i: (i, 0)),
          pl.BlockSpec((bn, DEG_W), lambda i: (i, 0)),
          pl.BlockSpec((bn, DEG_W), lambda i: (i + nb, 0)),
          pl.BlockSpec(w2.shape, lambda i: (0, 0)),
          pl.BlockSpec((1, h), lambda i: (0, 0)),
      ],
      out_specs=pl.BlockSpec((bn, h), lambda i: (i, 0)),
      out_shape=jax.ShapeDtypeStruct((n, h), jnp.float32),
  )(agg, agg, h1p, degp, degp, w2, b1.reshape(1, h))


def _tc_post(agg, h2p, degp, b2, batch3, f1, fb1, f2, fb2, n, bn, g):
  ng = n // bn
  nb = ng
  h = h2p.shape[1]
  out = f2.shape[1]

  def body(a0_ref, a1_ref, hp_ref, d0_ref, d1_ref, b_ref, bt_ref,
           f1_ref, fb1_ref, f2_ref, fb2_ref, z_ref, sums, cnt):
    i = pl.program_id(0)
    dinv = _dinv_blk(d0_ref, d1_ref)
    h2 = (a0_ref[...] + a1_ref[...] + hp_ref[...]) * dinv + b_ref[...]
    bt = bt_ref[0]
    gids = lax.broadcasted_iota(jnp.int32, (g, bn), 0)
    oh = jnp.where(bt == gids, 1.0, 0.0).astype(jnp.float32)
    part = jnp.dot(oh, h2, preferred_element_type=jnp.float32,
                   precision=lax.Precision.HIGHEST)
    pcnt = jnp.sum(oh, axis=1, keepdims=True)

    @pl.when(i == 0)
    def _():
      sums[...] = part
      cnt[...] = pcnt

    @pl.when(i > 0)
    def _():
      sums[...] += part
      cnt[...] += pcnt

    @pl.when(i == ng - 1)
    def _():
      gr = jnp.maximum(sums[...] / jnp.maximum(cnt[...], 1.0), 0.0)
      z1 = jnp.dot(gr, f1_ref[...], preferred_element_type=jnp.float32,
                   precision=lax.Precision.HIGHEST) + fb1_ref[...]
      z1 = jnp.maximum(z1, 0.0)
      z_ref[...] = jnp.dot(z1, f2_ref[...],
                           preferred_element_type=jnp.float32,
                           precision=lax.Precision.HIGHEST) + fb2_ref[...]

  return pl.pallas_call(
      body,
      grid=(ng,),
      in_specs=[
          pl.BlockSpec((bn, h), lambda i: (i, 0)),
          pl.BlockSpec((bn, h), lambda i: (i + nb, 0)),
          pl.BlockSpec((bn, h), lambda i: (i, 0)),
          pl.BlockSpec((bn, DEG_W), lambda i: (i, 0)),
          pl.BlockSpec((bn, DEG_W), lambda i: (i + nb, 0)),
          pl.BlockSpec((1, h), lambda i: (0, 0)),
          pl.BlockSpec((1, 1, bn), lambda i: (i, 0, 0)),
          pl.BlockSpec(f1.shape, lambda i: (0, 0)),
          pl.BlockSpec((1, h), lambda i: (0, 0)),
          pl.BlockSpec(f2.shape, lambda i: (0, 0)),
          pl.BlockSpec((1, out), lambda i: (0, 0)),
      ],
      out_specs=pl.BlockSpec((g, out), lambda i: (0, 0)),
      out_shape=jax.ShapeDtypeStruct((g, out), jnp.float32),
      scratch_shapes=[
          pltpu.VMEM((g, h), jnp.float32),
          pltpu.VMEM((g, 1), jnp.float32),
      ],
  )(agg, agg, h2p, degp, degp, b2.reshape(1, h), batch3,
    f1, fb1.reshape(1, h), f2, fb2.reshape(1, out))


def kernel(x, edge_index, batch, W1, b1, W2, b2, F1, fb1, F2, fb2):
  n, d = x.shape
  e = edge_index.shape[1]
  g = 128
  bn = 2000
  src = edge_index[0]
  dst = edge_index[1]
  batch3 = batch.reshape(n // bn, 1, bn)

  degp = _sc_deg(dst, n, e)
  h1p = _tc_pre(x, W1, degp, n, bn)
  a1 = _sc_agg(h1p, src, dst, n, e, d)
  h2p = _tc_mid(a1, h1p, degp, W2, b1, n, bn)
  a2 = _sc_agg(h2p, src, dst, n, e, d)
  z = _tc_post(a2, h2p, degp, b2, batch3, F1, fb1, F2, fb2, n, bn, g)
  return z
